# Initial kernel scaffold; baseline (speedup 1.0000x reference)
import jax
import jax.numpy as jnp
from jax.experimental import pallas as pl


def kernel(nf, di, dout, ae, de, doe, gt):
    raise NotImplementedError("write your pallas kernel here")


# speedup vs baseline: 6.5717x; 6.5717x over previous
import jax
import jax.numpy as jnp
import numpy as np
from jax import lax
from jax.experimental import pallas as pl
from jax.experimental.pallas import tpu as pltpu
from jax.experimental.pallas import tpu_sc as plsc

NG, NN, NFEAT = 256, 512, 9
HIDDEN = 128
NC, NS = 2, 16
NW = NC * NS
GPW = NG // NW
AROWS = NN * NFEAT // HIDDEN
DROWS = NN // HIDDEN
OUT_ROWS = NG * (NN + 1)


def _body(nf2, di2, dout2, ae, de, doe, gt, arep, ident, out,
          idx_v, dio_v, arep_v, ident_v, gbuf, dbuf, gt_v, accs, sem):
    cid = lax.axis_index("c")
    sid = lax.axis_index("s")
    wid = cid * NS + sid

    pltpu.sync_copy(arep, arep_v)
    pltpu.sync_copy(ident, ident_v)
    pltpu.sync_copy(gt, gt_v)
    base = sid * NN

    def _off_row(r, ref):
        def _off_lane(l, _):
            sl = (r, pl.ds(l * 16, 16))
            ref[sl] = ref[sl] + base
            return 0
        return lax.fori_loop(0, HIDDEN // 16, _off_lane, 0)

    def _off_arep(r, _):
        _off_row(r, arep_v)
        return 0

    def _off_ident(r, _):
        _off_row(r, ident_v)
        return 0

    lax.fori_loop(0, AROWS, _off_arep, 0)
    lax.fori_loop(0, DROWS, _off_ident, 0)

    def _graph(gi, _):
        g = wid * GPW + gi
        pltpu.sync_copy(nf2.at[pl.ds(g * AROWS, AROWS)], idx_v)
        pltpu.sync_copy(di2.at[pl.ds(g * DROWS, DROWS)], dio_v.at[pl.ds(0, DROWS)])
        pltpu.sync_copy(dout2.at[pl.ds(g * DROWS, DROWS)], dio_v.at[pl.ds(DROWS, DROWS)])

        def _de(r, _):
            pltpu.async_copy(de.at[dio_v.at[r]], dbuf, sem).wait()
            pltpu.sync_copy(dbuf, accs.at[pl.ds(base + r * HIDDEN, HIDDEN)])
            return 0
        lax.fori_loop(0, DROWS, _de, 0)

        def _doe(r, _):
            pltpu.async_copy(doe.at[dio_v.at[DROWS + r]], dbuf, sem).wait()
            pltpu.sync_copy(dbuf, accs.at[ident_v.at[r]], add=True)
            return 0
        lax.fori_loop(0, DROWS, _doe, 0)

        def _atom(r, _):
            pltpu.async_copy(ae.at[idx_v.at[r]], gbuf, sem).wait()
            pltpu.sync_copy(gbuf, accs.at[arep_v.at[r]], add=True)
            return 0
        lax.fori_loop(0, AROWS, _atom, 0)

        pltpu.sync_copy(gt_v, out.at[pl.ds(g * (NN + 1), 1)])
        pltpu.sync_copy(accs.at[pl.ds(base, NN)],
                        out.at[pl.ds(g * (NN + 1) + 1, NN)])
        return 0

    lax.fori_loop(0, GPW, _graph, 0)


_AREP = (np.arange(NN * NFEAT) // NFEAT).reshape(AROWS, HIDDEN).astype(np.int32)
_IDENT = np.arange(NN).reshape(DROWS, HIDDEN).astype(np.int32)

_sc_call = pl.kernel(
    _body,
    out_type=jax.ShapeDtypeStruct((OUT_ROWS, HIDDEN), jnp.float32),
    mesh=plsc.VectorSubcoreMesh(core_axis_name="c", subcore_axis_name="s",
                                num_cores=NC, num_subcores=NS),
    scratch_types=[
        pltpu.VMEM((AROWS, HIDDEN), jnp.int32),
        pltpu.VMEM((2 * DROWS, HIDDEN), jnp.int32),
        pltpu.VMEM((AROWS, HIDDEN), jnp.int32),
        pltpu.VMEM((DROWS, HIDDEN), jnp.int32),
        pltpu.VMEM((HIDDEN, HIDDEN), jnp.float32),
        pltpu.VMEM((HIDDEN, HIDDEN), jnp.float32),
        pltpu.VMEM((1, HIDDEN), jnp.float32),
        pltpu.VMEM_SHARED((NS * NN, HIDDEN), jnp.float32),
        pltpu.SemaphoreType.DMA,
    ],
    compiler_params=pltpu.CompilerParams(use_tc_tiling_on_sc=False),
)


@jax.jit
def kernel(nf, di, dout, ae, de, doe, gt):
    nf2 = nf.astype(jnp.int32).reshape(NG * AROWS, HIDDEN)
    di2 = di.astype(jnp.int32).reshape(NG * DROWS, HIDDEN)
    dout2 = dout.astype(jnp.int32).reshape(NG * DROWS, HIDDEN)
    out = _sc_call(nf2, di2, dout2, ae, de, doe, gt,
                   jnp.asarray(_AREP), jnp.asarray(_IDENT))
    return out.reshape(NG, NN + 1, HIDDEN)


# speedup vs baseline: 7.8389x; 1.1928x over previous
import jax
import jax.numpy as jnp
import numpy as np
from jax import lax
from jax.experimental import pallas as pl
from jax.experimental.pallas import tpu as pltpu
from jax.experimental.pallas import tpu_sc as plsc

NG, NN, NFEAT = 256, 512, 9
HIDDEN = 128
NC, NS = 2, 16
NW = NC * NS
UNITS = 2 * NG
UPW = UNITS // NW
NODES_U = NN // 2
AROWS = NODES_U * NFEAT // HIDDEN
DROWS = NODES_U // HIDDEN
OUT_ROWS = NG * (NN + 1)
NBUF = 3


def _body(nf2, di2, dout2, ae, de, doe, gt, arep, ident, out,
          idx_v, dio_v, arep_v, ident_v, gb0, gb1, gb2, gt_v, accs,
          gs0, gs1, gs2, ss0, ss1, ss2):
    bufs = ((gb0, gs0, ss0), (gb1, gs1, ss1), (gb2, gs2, ss2))
    cid = lax.axis_index("c")
    sid = lax.axis_index("s")
    wid = cid * NS + sid
    base = sid * NODES_U

    pltpu.sync_copy(arep, arep_v)
    pltpu.sync_copy(ident, ident_v)
    pltpu.sync_copy(gt, gt_v)

    def _off(ref, nrow):
        def _row(r, _):
            def _lane(l, _):
                sl = (r, pl.ds(l * 16, 16))
                ref[sl] = ref[sl] + base
                return 0
            return lax.fori_loop(0, HIDDEN // 16, _lane, 0)
        lax.fori_loop(0, nrow, _row, 0)

    _off(arep_v, AROWS)
    _off(ident_v, DROWS)

    def _unit(ui, _):
        u = wid * UPW + ui
        g = u // 2
        h = u - g * 2
        pltpu.sync_copy(nf2.at[pl.ds(u * AROWS, AROWS)], idx_v)
        pltpu.sync_copy(di2.at[pl.ds(u * DROWS, DROWS)],
                        dio_v.at[pl.ds(0, DROWS)])
        pltpu.sync_copy(dout2.at[pl.ds(u * DROWS, DROWS)],
                        dio_v.at[pl.ds(DROWS, DROWS)])

        for r in range(DROWS):
            pltpu.async_copy(de.at[dio_v.at[r]], gb0, gs0).wait()
            pltpu.sync_copy(gb0, accs.at[pl.ds(base + r * HIDDEN, HIDDEN)])
        for r in range(DROWS):
            pltpu.async_copy(doe.at[dio_v.at[DROWS + r]], gb0, gs0).wait()
            pltpu.sync_copy(gb0, accs.at[ident_v.at[r]], add=True)

        for par in range(2):
            npk = AROWS // 2 // NBUF

            for b, (gb, gs, _) in enumerate(bufs):
                pltpu.async_copy(ae.at[idx_v.at[par + 2 * b]], gb, gs)

            def _pipe(t, _, par=par):
                k0 = t * NBUF
                for b, (gb, gs, ss) in enumerate(bufs):
                    r = par + 2 * (k0 + b)
                    pltpu.make_async_copy(ae.at[idx_v.at[r]], gb, gs).wait()
                    pltpu.async_copy(gb, accs.at[arep_v.at[r]], ss, add=True)
                for b, (gb, gs, ss) in enumerate(bufs):
                    r = par + 2 * (k0 + b)
                    pltpu.make_async_copy(gb, accs.at[arep_v.at[r]], ss).wait()
                    pltpu.async_copy(ae.at[idx_v.at[r + 2 * NBUF]], gb, gs)
                return 0

            lax.fori_loop(0, npk - 1, _pipe, 0)

            for b, (gb, gs, ss) in enumerate(bufs):
                r = par + 2 * ((npk - 1) * NBUF + b)
                pltpu.make_async_copy(ae.at[idx_v.at[r]], gb, gs).wait()
                pltpu.async_copy(gb, accs.at[arep_v.at[r]], ss, add=True)
            for b, (gb, gs, ss) in enumerate(bufs):
                r = par + 2 * ((npk - 1) * NBUF + b)
                pltpu.make_async_copy(gb, accs.at[arep_v.at[r]], ss).wait()

        pltpu.sync_copy(accs.at[pl.ds(base, NODES_U)],
                        out.at[pl.ds(g * (NN + 1) + 1 + h * NODES_U, NODES_U)])

        @pl.when(h == 0)
        def _():
            pltpu.sync_copy(gt_v, out.at[pl.ds(g * (NN + 1), 1)])
        return 0

    lax.fori_loop(0, UPW, _unit, 0)


_AREP = (np.arange(NODES_U * NFEAT) // NFEAT).reshape(AROWS, HIDDEN).astype(np.int32)
_IDENT = np.arange(NODES_U).reshape(DROWS, HIDDEN).astype(np.int32)

_sc_call = pl.kernel(
    _body,
    out_type=jax.ShapeDtypeStruct((OUT_ROWS, HIDDEN), jnp.float32),
    mesh=plsc.VectorSubcoreMesh(core_axis_name="c", subcore_axis_name="s",
                                num_cores=NC, num_subcores=NS),
    scratch_types=[
        pltpu.VMEM((AROWS, HIDDEN), jnp.int32),
        pltpu.VMEM((2 * DROWS, HIDDEN), jnp.int32),
        pltpu.VMEM((AROWS, HIDDEN), jnp.int32),
        pltpu.VMEM((DROWS, HIDDEN), jnp.int32),
        pltpu.VMEM((HIDDEN, HIDDEN), jnp.float32),
        pltpu.VMEM((HIDDEN, HIDDEN), jnp.float32),
        pltpu.VMEM((HIDDEN, HIDDEN), jnp.float32),
        pltpu.VMEM((1, HIDDEN), jnp.float32),
        pltpu.VMEM_SHARED((NS * NODES_U, HIDDEN), jnp.float32),
        pltpu.SemaphoreType.DMA,
        pltpu.SemaphoreType.DMA,
        pltpu.SemaphoreType.DMA,
        pltpu.SemaphoreType.DMA,
        pltpu.SemaphoreType.DMA,
        pltpu.SemaphoreType.DMA,
    ],
    compiler_params=pltpu.CompilerParams(use_tc_tiling_on_sc=False),
)


@jax.jit
def kernel(nf, di, dout, ae, de, doe, gt):
    nf2 = nf.astype(jnp.int32).reshape(UNITS * AROWS, HIDDEN)
    di2 = di.astype(jnp.int32).reshape(UNITS * DROWS, HIDDEN)
    dout2 = dout.astype(jnp.int32).reshape(UNITS * DROWS, HIDDEN)
    out = _sc_call(nf2, di2, dout2, ae, de, doe, gt,
                   jnp.asarray(_AREP), jnp.asarray(_IDENT))
    return out.reshape(NG, NN + 1, HIDDEN)


# speedup vs baseline: 7.9320x; 1.0119x over previous
import jax
import jax.numpy as jnp
import numpy as np
from jax import lax
from jax.experimental import pallas as pl
from jax.experimental.pallas import tpu as pltpu
from jax.experimental.pallas import tpu_sc as plsc

NG, NN, NFEAT = 256, 512, 9
HIDDEN = 128
NC, NS = 2, 16
NW = NC * NS
UNITS = 2 * NG
UPW = UNITS // NW
NODES_U = NN // 2
AROWS = NODES_U * NFEAT // HIDDEN
DROWS = NODES_U // HIDDEN
NSTREAM = 2 * DROWS + AROWS
OUT_ROWS = NG * (NN + 1)


def _atom_row(k):
    a = k - 2 * DROWS
    return jnp.where(a < AROWS // 2, 2 * a, 2 * a - (AROWS - 1))


def _body(nf2, di2, dout2, ae, de, doe, gt, arep, ident, out,
          idx_v, dio_v, arep_v, ident_v, gb0, gb1, gt_v, accs,
          gs0, gs1, ss0, ss1, isem, osem):
    bufs = ((gb0, gs0, ss0), (gb1, gs1, ss1))
    cid = lax.axis_index("c")
    sid = lax.axis_index("s")
    wid = cid * NS + sid

    pltpu.sync_copy(arep, arep_v)
    pltpu.sync_copy(ident, ident_v)
    pltpu.sync_copy(gt, gt_v)

    def _prefetch(u, pb):
        pltpu.async_copy(nf2.at[pl.ds(u * AROWS, AROWS)], idx_v.at[pb], isem)
        pltpu.async_copy(di2.at[pl.ds(u * DROWS, DROWS)],
                         dio_v.at[pb, pl.ds(0, DROWS)], isem)
        pltpu.async_copy(dout2.at[pl.ds(u * DROWS, DROWS)],
                         dio_v.at[pb, pl.ds(DROWS, DROWS)], isem)

    def _wait_prefetch(pb):
        pltpu.make_async_copy(nf2.at[pl.ds(0, AROWS)], idx_v.at[pb], isem).wait()
        pltpu.make_async_copy(di2.at[pl.ds(0, DROWS)],
                              dio_v.at[pb, pl.ds(0, DROWS)], isem).wait()
        pltpu.make_async_copy(dout2.at[pl.ds(0, DROWS)],
                              dio_v.at[pb, pl.ds(DROWS, DROWS)], isem).wait()

    def _issue_gather(k, pb, gb, gs):
        @pl.when(k < DROWS)
        def _():
            pltpu.async_copy(de.at[dio_v.at[pb, k]], gb, gs)

        @pl.when(jnp.logical_and(k >= DROWS, k < 2 * DROWS))
        def _():
            pltpu.async_copy(doe.at[dio_v.at[pb, k]], gb, gs)

        @pl.when(k >= 2 * DROWS)
        def _():
            pltpu.async_copy(ae.at[idx_v.at[pb, _atom_row(k)]], gb, gs)

    def _issue_scatter(k, hb, gb, ss):
        av = accs.at[hb, sid]

        @pl.when(k < DROWS)
        def _():
            pltpu.async_copy(gb, av.at[pl.ds(k * HIDDEN, HIDDEN)], ss)

        @pl.when(jnp.logical_and(k >= DROWS, k < 2 * DROWS))
        def _():
            pltpu.async_copy(gb, av.at[ident_v.at[k - DROWS]], ss, add=True)

        @pl.when(k >= 2 * DROWS)
        def _():
            pltpu.async_copy(gb, av.at[arep_v.at[_atom_row(k)]], ss, add=True)

    def _wait64k(sem, gb):
        pltpu.make_async_copy(ae.at[pl.ds(0, HIDDEN)], gb, sem).wait()

    _prefetch(wid * UPW, 0)

    def _unit(ui, _):
        u = wid * UPW + ui
        g = u // 2
        h = u - g * 2
        pb = ui % 2
        hb = ui % 2
        _wait_prefetch(pb)

        @pl.when(ui + 1 < UPW)
        def _():
            _prefetch(u + 1, 1 - pb)

        for b, (gb, gs, _) in enumerate(bufs):
            _issue_gather(jnp.int32(b), pb, gb, gs)

        def _pipe(t, _):
            k0 = 2 * t
            for b, (gb, gs, ss) in enumerate(bufs):
                _wait64k(gs, gb)
                _issue_scatter(k0 + b, hb, gb, ss)
            for b, (gb, gs, ss) in enumerate(bufs):
                _wait64k(ss, gb)
                _issue_gather(k0 + b + 2, pb, gb, gs)
            return 0

        lax.fori_loop(0, NSTREAM // 2 - 1, _pipe, 0)

        for b, (gb, gs, ss) in enumerate(bufs):
            _wait64k(gs, gb)
            _issue_scatter(jnp.int32(NSTREAM - 2 + b), hb, gb, ss)
        for b, (gb, gs, ss) in enumerate(bufs):
            _wait64k(ss, gb)

        @pl.when(ui > 0)
        def _():
            pltpu.make_async_copy(ae.at[pl.ds(0, NODES_U)],
                                  accs.at[hb, sid], osem).wait()

        pltpu.async_copy(accs.at[hb, sid],
                         out.at[pl.ds(g * (NN + 1) + 1 + h * NODES_U, NODES_U)],
                         osem)

        @pl.when(h == 0)
        def _():
            pltpu.sync_copy(gt_v, out.at[pl.ds(g * (NN + 1), 1)])
        return 0

    lax.fori_loop(0, UPW, _unit, 0)
    pltpu.make_async_copy(ae.at[pl.ds(0, NODES_U)],
                          accs.at[(UPW - 1) % 2, sid], osem).wait()


_AREP = (np.arange(NODES_U * NFEAT) // NFEAT).reshape(AROWS, HIDDEN).astype(np.int32)
_IDENT = np.arange(NODES_U).reshape(DROWS, HIDDEN).astype(np.int32)

_sc_call = pl.kernel(
    _body,
    out_type=jax.ShapeDtypeStruct((OUT_ROWS, HIDDEN), jnp.float32),
    mesh=plsc.VectorSubcoreMesh(core_axis_name="c", subcore_axis_name="s",
                                num_cores=NC, num_subcores=NS),
    scratch_types=[
        pltpu.VMEM((2, AROWS, HIDDEN), jnp.int32),
        pltpu.VMEM((2, 2 * DROWS, HIDDEN), jnp.int32),
        pltpu.VMEM((AROWS, HIDDEN), jnp.int32),
        pltpu.VMEM((DROWS, HIDDEN), jnp.int32),
        pltpu.VMEM((HIDDEN, HIDDEN), jnp.float32),
        pltpu.VMEM((HIDDEN, HIDDEN), jnp.float32),
        pltpu.VMEM((1, HIDDEN), jnp.float32),
        pltpu.VMEM_SHARED((2, NS, NODES_U, HIDDEN), jnp.float32),
        pltpu.SemaphoreType.DMA,
        pltpu.SemaphoreType.DMA,
        pltpu.SemaphoreType.DMA,
        pltpu.SemaphoreType.DMA,
        pltpu.SemaphoreType.DMA,
        pltpu.SemaphoreType.DMA,
    ],
    compiler_params=pltpu.CompilerParams(use_tc_tiling_on_sc=False),
)


@jax.jit
def kernel(nf, di, dout, ae, de, doe, gt):
    nf2 = nf.astype(jnp.int32).reshape(UNITS * AROWS, HIDDEN)
    di2 = di.astype(jnp.int32).reshape(UNITS * DROWS, HIDDEN)
    dout2 = dout.astype(jnp.int32).reshape(UNITS * DROWS, HIDDEN)
    out = _sc_call(nf2, di2, dout2, ae, de, doe, gt,
                   jnp.asarray(_AREP), jnp.asarray(_IDENT))
    return out.reshape(NG, NN + 1, HIDDEN)
